# Initial kernel scaffold; baseline (speedup 1.0000x reference)
import jax
import jax.numpy as jnp
from jax.experimental import pallas as pl


def kernel(hidden_states, ln_gamma, ln_beta, router_W, router_b, up_W, up_b, down_W, down_b):
    raise NotImplementedError("write your pallas kernel here")


# speedup vs baseline: 3.3774x; 3.3774x over previous
import functools

import jax
import jax.numpy as jnp
from jax import lax
from jax.experimental import pallas as pl
from jax.experimental.pallas import tpu as pltpu
from jax.experimental.pallas import tpu_sc as plsc

_pallas_call = pl.pallas_call

_S = 2048
_D = 1024
_I = 2048
_E = 8
_K = 2
_GT = 512
_NT = _S * _K // _GT + _E
_P = _NT * _GT

_NW = 32
_RPW = _P // _NW
_GCH = 48
_TPW = _S // _NW
_CCH = 16


def _gelu_exact(x):
    inner = 0.7978845608028654 * x * (1.0 + 0.044715 * x * x)
    return 0.5 * x * (1.0 + jnp.tanh(inner))


def _cumsum_rows(a):
    n = a.shape[0]
    sh = 1
    while sh < n:
        a = a + jnp.concatenate(
            [jnp.zeros((sh, a.shape[1]), a.dtype), a[:-sh, :]], axis=0)
        sh *= 2
    return a


def _excl_cumsum_lanes(a):
    a = jnp.concatenate([jnp.zeros((1, 1), a.dtype), a[:, :-1]], axis=1)
    sh = 1
    while sh < a.shape[1]:
        a = a + jnp.concatenate(
            [jnp.zeros((1, sh), a.dtype), a[:, :-sh]], axis=1)
        sh *= 2
    return a


def _router_body(x_ref, g_ref, b_ref, w_ref, rb_ref,
                 pos1_ref, pos2_ref, w1_ref, w2_ref, te_ref, ntc_ref):
    x = x_ref[...]
    mu = jnp.mean(x, axis=1, keepdims=True)
    var = jnp.mean((x - mu) ** 2, axis=1, keepdims=True)
    xn = (x - mu) * lax.rsqrt(var + 1e-5) * g_ref[...] + b_ref[...]
    logits = lax.dot_general(xn, w_ref[...], (((1,), (1,)), ((), ())),
                             preferred_element_type=jnp.float32)
    logits = logits + rb_ref[...]
    idx = lax.broadcasted_iota(jnp.int32, logits.shape, 1)
    m1 = jnp.max(logits, axis=1, keepdims=True)
    a1 = jnp.min(jnp.where(logits == m1, idx, _E), axis=1, keepdims=True)
    l2 = jnp.where(idx == a1, -jnp.inf, logits)
    m2 = jnp.max(l2, axis=1, keepdims=True)
    a2 = jnp.min(jnp.where(l2 == m2, idx, _E), axis=1, keepdims=True)
    w1 = 1.0 / (1.0 + jnp.exp(m2 - m1))
    w2 = 1.0 - w1

    oh1 = (idx == a1).astype(jnp.float32)
    oh2 = (idx == a2).astype(jnp.float32)
    cs1 = _cumsum_rows(oh1)
    cs2 = _cumsum_rows(oh2)
    n1 = cs1[_S - 1:_S, :]
    n2 = cs2[_S - 1:_S, :]
    nt = jnp.ceil((n1 + n2) * (1.0 / _GT))
    toff = _excl_cumsum_lanes(nt)

    r1 = jnp.sum((cs1 - oh1) * oh1, axis=1, keepdims=True)
    r2 = (jnp.sum((cs2 - oh2) * oh2, axis=1, keepdims=True)
          + jnp.sum(n1 * oh2, axis=1, keepdims=True))
    base1 = _GT * jnp.sum(toff * oh1, axis=1, keepdims=True)
    base2 = _GT * jnp.sum(toff * oh2, axis=1, keepdims=True)
    pos1_ref[...] = (base1 + r1).astype(jnp.int32)
    pos2_ref[...] = (base2 + r2).astype(jnp.int32)
    w1_ref[...] = w1
    w2_ref[...] = w2

    eye = (lax.broadcasted_iota(jnp.int32, (_E, _E), 0)
           == lax.broadcasted_iota(jnp.int32, (_E, _E), 1)).astype(jnp.float32)
    toff_col = lax.dot_general(eye, toff, (((1,), (1,)), ((), ())),
                               preferred_element_type=jnp.float32)
    jrow = lax.broadcasted_iota(jnp.int32, (1, _NT), 1).astype(jnp.float32)
    te = jnp.sum((toff_col <= jrow).astype(jnp.float32), axis=0,
                 keepdims=True) - 1.0
    te_ref[...] = jnp.clip(te, 0.0, float(_E - 1)).astype(jnp.int32)
    ntc_ref[...] = jnp.sum(nt, axis=1, keepdims=True).astype(jnp.int32)


def _router_call(x, g, b, rw, rb):
    return _pallas_call(
        _router_body,
        out_shape=[
            jax.ShapeDtypeStruct((_S, 1), jnp.int32),
            jax.ShapeDtypeStruct((_S, 1), jnp.int32),
            jax.ShapeDtypeStruct((_S, 1), jnp.float32),
            jax.ShapeDtypeStruct((_S, 1), jnp.float32),
            jax.ShapeDtypeStruct((1, _NT), jnp.int32),
            jax.ShapeDtypeStruct((1, 1), jnp.int32),
        ],
    )(x, g, b, rw, rb)


@functools.lru_cache(maxsize=None)
def _dispatch_call():
    mesh = plsc.VectorSubcoreMesh(core_axis_name="c", subcore_axis_name="s")
    return pl.kernel(
        _dispatch_body,
        mesh=mesh,
        out_type=[
            jax.ShapeDtypeStruct((_P, _D), jnp.float32),
            jax.ShapeDtypeStruct((_P,), jnp.float32),
        ],
        scratch_types=[
            pltpu.VMEM((_TPW,), jnp.int32),
            pltpu.VMEM((_TPW,), jnp.int32),
            pltpu.VMEM((_S,), jnp.int32),
            pltpu.VMEM((_S,), jnp.int32),
            pltpu.VMEM((_S,), jnp.float32),
            pltpu.VMEM((_S,), jnp.float32),
            pltpu.VMEM((_P,), jnp.float32),
            pltpu.VMEM((_TPW, _D), jnp.float32),
            pltpu.SemaphoreType.DMA,
            pltpu.SemaphoreType.DMA,
        ],
        compiler_params=pltpu.CompilerParams(needs_layout_passes=False),
    )


def _dispatch_body(pos1_h, pos2_h, w1_h, w2_h, x_h, xp_h, ws_h,
                   p1_v, p2_v, pos1_v, pos2_v, w1_v, w2_v, ws_v, xbuf_v,
                   semw1, semw2):
    c = lax.axis_index("c")
    s = lax.axis_index("s")
    wid = s * 2 + c
    base = wid * _TPW
    pltpu.sync_copy(x_h.at[pl.ds(base, _TPW)], xbuf_v)
    pltpu.sync_copy(pos1_h.at[pl.ds(base, _TPW)], p1_v)
    pltpu.sync_copy(pos2_h.at[pl.ds(base, _TPW)], p2_v)
    pltpu.async_copy(xbuf_v, xp_h.at[p1_v], semw1)
    pltpu.async_copy(xbuf_v, xp_h.at[p2_v], semw2)

    @pl.when(jnp.logical_and(s == 0, c == 0))
    def _sorted_weights():
        pltpu.sync_copy(pos1_h, pos1_v)
        pltpu.sync_copy(pos2_h, pos2_v)
        pltpu.sync_copy(w1_h, w1_v)
        pltpu.sync_copy(w2_h, w2_v)

        def scat(i, carry):
            sl = pl.ds(i * 16, 16)
            plsc.store_scatter(ws_v, [pos1_v[sl]], w1_v[sl])
            plsc.store_scatter(ws_v, [pos2_v[sl]], w2_v[sl])
            return carry

        lax.fori_loop(0, _S // 16, scat, 0)
        pltpu.sync_copy(ws_v, ws_h)

    pltpu.make_async_copy(xbuf_v, xp_h.at[p1_v], semw1).wait()
    pltpu.make_async_copy(xbuf_v, xp_h.at[p2_v], semw2).wait()


def _ffn_body(te_ref, nt_ref, x_ref, w_ref, uw_ref, ub_ref, dw_ref, db_ref,
              y_ref):
    j = pl.program_id(0)

    @pl.when(j < nt_ref[0])
    def _():
        xs = x_ref[...]
        h = lax.dot_general(xs, uw_ref[0], (((1,), (1,)), ((), ())),
                            preferred_element_type=jnp.float32)
        h = _gelu_exact(h + ub_ref[0])
        y = lax.dot_general(h, dw_ref[0], (((1,), (1,)), ((), ())),
                            preferred_element_type=jnp.float32)
        y_ref[...] = (y + db_ref[0]) * w_ref[...]


def _ffn_call(te, ntc, xp, ws2, uw, ub, dw, db):
    grid_spec = pltpu.PrefetchScalarGridSpec(
        num_scalar_prefetch=2,
        grid=(_NT,),
        in_specs=[
            pl.BlockSpec((_GT, _D), lambda j, te, nt: (j, 0)),
            pl.BlockSpec((_GT, 1), lambda j, te, nt: (j, 0)),
            pl.BlockSpec((1, _I, _D), lambda j, te, nt: (te[j], 0, 0)),
            pl.BlockSpec((1, 1, _I), lambda j, te, nt: (te[j], 0, 0)),
            pl.BlockSpec((1, _D, _I), lambda j, te, nt: (te[j], 0, 0)),
            pl.BlockSpec((1, 1, _D), lambda j, te, nt: (te[j], 0, 0)),
        ],
        out_specs=pl.BlockSpec((_GT, _D), lambda j, te, nt: (j, 0)),
    )
    return _pallas_call(
        _ffn_body,
        grid_spec=grid_spec,
        out_shape=jax.ShapeDtypeStruct((_P, _D), jnp.float32),
        compiler_params=pltpu.CompilerParams(
            dimension_semantics=("arbitrary",),
        ),
    )(te, ntc, xp, ws2, uw, ub.reshape(_E, 1, _I), dw, db.reshape(_E, 1, _D))


@functools.lru_cache(maxsize=None)
def _combine_call():
    mesh = plsc.VectorSubcoreMesh(core_axis_name="c", subcore_axis_name="s")
    return pl.kernel(
        _combine_body,
        mesh=mesh,
        out_type=jax.ShapeDtypeStruct((_S, _D), jnp.float32),
        scratch_types=[
            pltpu.VMEM((_TPW,), jnp.int32),
            pltpu.VMEM((_TPW,), jnp.int32),
            pltpu.VMEM((_CCH, _D), jnp.float32),
            pltpu.VMEM((_CCH, _D), jnp.float32),
            pltpu.VMEM((_CCH, _D), jnp.float32),
            pltpu.VMEM((_CCH, _D), jnp.float32),
            pltpu.VMEM((_CCH, _D), jnp.float32),
            pltpu.VMEM((_CCH, _D), jnp.float32),
            pltpu.SemaphoreType.DMA,
            pltpu.SemaphoreType.DMA,
            pltpu.SemaphoreType.DMA,
            pltpu.SemaphoreType.DMA,
            pltpu.SemaphoreType.DMA,
            pltpu.SemaphoreType.DMA,
            pltpu.SemaphoreType.DMA,
            pltpu.SemaphoreType.DMA,
            pltpu.SemaphoreType.DMA,
        ],
        compiler_params=pltpu.CompilerParams(needs_layout_passes=False),
    )


def _combine_body(y_h, pos1_h, pos2_h, out_h,
                  p1_v, p2_v, a0_v, a1_v, a2_v, b0_v, b1_v, b2_v,
                  sa0, sa1, sa2, sb0, sb1, sb2, sw0, sw1, sw2):
    c = lax.axis_index("c")
    s = lax.axis_index("s")
    wid = s * 2 + c
    base = wid * _TPW
    pltpu.sync_copy(pos1_h.at[pl.ds(base, _TPW)], p1_v)
    pltpu.sync_copy(pos2_h.at[pl.ds(base, _TPW)], p2_v)
    nch = _TPW // _CCH
    nbuf = 3
    abuf = (a0_v, a1_v, a2_v)
    bbuf = (b0_v, b1_v, b2_v)
    sa = (sa0, sa1, sa2)
    sb = (sb0, sb1, sb2)
    sw = (sw0, sw1, sw2)

    def _fire_gather(ch):
        k = ch % nbuf
        pltpu.async_copy(y_h.at[p1_v.at[pl.ds(ch * _CCH, _CCH)]],
                         abuf[k], sa[k])
        pltpu.async_copy(y_h.at[p2_v.at[pl.ds(ch * _CCH, _CCH)]],
                         bbuf[k], sb[k])

    def _wait_gather(ch):
        k = ch % nbuf
        pltpu.make_async_copy(y_h.at[p1_v.at[pl.ds(ch * _CCH, _CCH)]],
                              abuf[k], sa[k]).wait()
        pltpu.make_async_copy(y_h.at[p2_v.at[pl.ds(ch * _CCH, _CCH)]],
                              bbuf[k], sb[k]).wait()

    def _fire_write(ch):
        k = ch % nbuf
        pltpu.async_copy(abuf[k], out_h.at[pl.ds(base + ch * _CCH, _CCH)],
                         sw[k])

    def _wait_write(ch):
        k = ch % nbuf
        pltpu.make_async_copy(abuf[k],
                              out_h.at[pl.ds(base + ch * _CCH, _CCH)],
                              sw[k]).wait()

    for ch in range(min(nbuf, nch)):
        _fire_gather(ch)
    for ch in range(nch):
        _wait_gather(ch)
        k = ch % nbuf
        av = abuf[k]
        bv = bbuf[k]

        def row(i, carry, av=av, bv=bv):
            def col(kk, carry2):
                sl = pl.ds(kk * 16, 16)
                av[i, sl] = av[i, sl] + bv[i, sl]
                return carry2

            lax.fori_loop(0, _D // 16, col, 0, unroll=16)
            return carry

        lax.fori_loop(0, _CCH, row, 0, unroll=2)
        _fire_write(ch)
        if ch + nbuf < nch:
            _wait_write(ch)
            _fire_gather(ch + nbuf)
    for ch in range(max(0, nch - nbuf), nch):
        _wait_write(ch)


def kernel(hidden_states, ln_gamma, ln_beta, router_W, router_b, up_W, up_b,
           down_W, down_b):
    B, S, D = hidden_states.shape
    x = hidden_states.reshape(S, D)
    pos1, pos2, w1, w2, te, ntc = _router_call(
        x, ln_gamma.reshape(1, D), ln_beta.reshape(1, D), router_W,
        router_b.reshape(1, _E))
    pos1f = pos1.reshape(_S)
    pos2f = pos2.reshape(_S)
    w1f = w1.reshape(_S)
    w2f = w2.reshape(_S)
    xp, ws = _dispatch_call()(pos1f, pos2f, w1f, w2f, x)
    y = _ffn_call(te.reshape(_NT), ntc.reshape(1), xp, ws.reshape(_P, 1),
                  up_W, up_b, down_W, down_b)
    out = _combine_call()(y, pos1f, pos2f)
    return out.reshape(B, S, D)
